# baseline (speedup 1.0000x reference)
import functools

import jax
import jax.numpy as jnp
from jax import lax
from jax.experimental import pallas as pl
from jax.experimental.pallas import tpu as pltpu
from jax.experimental.pallas import tpu_sc as plsc

N = 10000
E = 320000
D = 128
NC = 2
NS = 16
NW = NC * NS
B = 128
EP = 327680
EPW = EP // NW
NCHUNK = EPW // B
SC_CH = 16
NSUP = NCHUNK // SC_CH
NPAD = 10240
NPW = NPAD // NS
FV = D // 16
BLK = 2000


def _sc_segment_sum(y, src3, dst3, ew2):
    mesh = plsc.VectorSubcoreMesh(core_axis_name="c", subcore_axis_name="s")

    @functools.partial(
        pl.kernel,
        out_type=jax.ShapeDtypeStruct((NC, NPAD, D), jnp.float32),
        mesh=mesh,
        scratch_types=[
            pltpu.VMEM((SC_CH, B), jnp.int32),
            pltpu.VMEM((SC_CH, B), jnp.int32),
            pltpu.VMEM((SC_CH, B), jnp.float32),
            pltpu.VMEM((B, D), jnp.float32),
            pltpu.VMEM((B, D), jnp.float32),
            pltpu.VMEM_SHARED((NPAD, D), jnp.float32),
            pltpu.SemaphoreType.DMA,
            pltpu.SemaphoreType.DMA,
        ],
    )
    def seg(y_hbm, src_hbm, dst_hbm, ew_hbm, out_hbm,
            src_v, dst_v, ew_v, rows0_v, rows1_v, acc_sh, sem0, sem1):
        rows = [rows0_v, rows1_v]
        gsem = [sem0, sem1]
        cid = lax.axis_index("c")
        sid = lax.axis_index("s")
        wid = sid * NC + cid

        z16 = jnp.zeros((16,), jnp.float32)

        def zrow(i, carry):
            for f in range(FV):
                rows0_v[i, pl.ds(f * 16, 16)] = z16
            return carry

        lax.fori_loop(0, B, zrow, 0)
        for z in range(NPW // B):
            pltpu.sync_copy(rows0_v,
                            acc_sh.at[pl.ds((sid * (NPW // B) + z) * B, B)])

        plsc.subcore_barrier()

        def sup(s, scarry):
            pltpu.sync_copy(src_hbm.at[wid, s], src_v)
            pltpu.sync_copy(dst_hbm.at[wid, s], dst_v)
            pltpu.sync_copy(ew_hbm.at[wid, s], ew_v)

            pltpu.async_copy(y_hbm.at[src_v.at[0]], rows[0], gsem[0])

            def chunk(c, carry):
                for k in range(2):
                    @pl.when(c % 2 == k)
                    def _(k=k):
                        cur_v = rows[k]
                        nk = 1 - k

                        pltpu.make_async_copy(
                            y_hbm.at[pl.ds(0, B)], cur_v, gsem[k]).wait()

                        @pl.when(c < SC_CH - 1)
                        def _():
                            pltpu.async_copy(
                                y_hbm.at[src_v.at[c + 1]], rows[nk], gsem[nk])

                        for g in range(B // 16):
                            ew16 = ew_v[c, pl.ds(g * 16, 16)]
                            for j in range(16):
                                w = jnp.broadcast_to(ew16[j], (16,))
                                i = g * 16 + j
                                for f in range(FV):
                                    cur_v[i, pl.ds(f * 16, 16)] = (
                                        cur_v[i, pl.ds(f * 16, 16)] * w)

                        pltpu.sync_copy(cur_v, acc_sh.at[dst_v.at[c]], add=True)

                return carry

            lax.fori_loop(0, SC_CH, chunk, 0)
            return scarry

        lax.fori_loop(0, NSUP, sup, 0)

        plsc.subcore_barrier()
        pltpu.sync_copy(acc_sh.at[pl.ds(sid * NPW, NPW)],
                        out_hbm.at[cid, pl.ds(sid * NPW, NPW)])

    return seg(y, src3, dst3, ew2)


def _dotT(a, w):
    return lax.dot_general(a, w, (((1,), (1,)), ((), ())),
                           preferred_element_type=jnp.float32)


def _layer1_body(p_ref, x_ref, wr_ref, wo_ref, b_ref, h_ref):
    aggr = p_ref[0] + p_ref[1]
    h_ref[...] = jnp.maximum(
        _dotT(aggr, wr_ref[...]) + b_ref[...] + _dotT(x_ref[...], wo_ref[...]), 0.0)


def _layer2_body(p_ref, h_ref, wr_ref, wo_ref, b_ref, wl_ref, bl_ref, o_ref):
    aggr = p_ref[0] + p_ref[1]
    g = jnp.maximum(
        _dotT(aggr, wr_ref[...]) + b_ref[...] + _dotT(h_ref[...], wo_ref[...]), 0.0)
    o_ref[...] = _dotT(g, wl_ref[...]) + bl_ref[...]


_row_spec = pl.BlockSpec((BLK, D), lambda i: (i, 0))
_w_spec = pl.BlockSpec((D, D), lambda i: (0, 0))
_b_spec = pl.BlockSpec((1, D), lambda i: (0, 0))
_p_spec = pl.BlockSpec((NC, BLK, D), lambda i: (0, i, 0))
_nd_f32 = jax.ShapeDtypeStruct((N, D), jnp.float32)


def _tc_layer1(parts, x, W_rel, b_rel, W_root):
    return pl.pallas_call(
        _layer1_body,
        grid=(N // BLK,),
        in_specs=[_p_spec, _row_spec, _w_spec, _w_spec, _b_spec],
        out_specs=_row_spec,
        out_shape=_nd_f32,
    )(parts, x, W_rel, W_root, b_rel.reshape(1, D))


def _tc_layer2(parts, h, W_rel, b_rel, W_root, W_lin, b_lin):
    return pl.pallas_call(
        _layer2_body,
        grid=(N // BLK,),
        in_specs=[_p_spec, _row_spec, _w_spec, _w_spec, _b_spec, _w_spec, _b_spec],
        out_specs=_row_spec,
        out_shape=_nd_f32,
    )(parts, h, W_rel, W_root, b_rel.reshape(1, D), W_lin, b_lin.reshape(1, D))


def kernel(x, edge_index, edge_weight,
           W_rel1, b_rel1, W_root1, W_rel2, b_rel2, W_root2, W_lin, b_lin):
    zpad = jnp.zeros((EP - E,), jnp.int32)
    src3 = jnp.concatenate([edge_index[0].astype(jnp.int32), zpad]
                           ).reshape(NW, NSUP, SC_CH, B)
    dst3 = jnp.concatenate([edge_index[1].astype(jnp.int32), zpad]
                           ).reshape(NW, NSUP, SC_CH, B)
    ew2 = jnp.concatenate([edge_weight.astype(jnp.float32),
                           zpad.astype(jnp.float32)]
                          ).reshape(NW, NSUP, SC_CH, B)

    p1 = _sc_segment_sum(x, src3, dst3, ew2)
    h = _tc_layer1(p1, x, W_rel1, b_rel1, W_root1)
    p2 = _sc_segment_sum(h, src3, dst3, ew2)
    return _tc_layer2(p2, h, W_rel2, b_rel2, W_root2, W_lin, b_lin)


# speedup vs baseline: 2.5153x; 2.5153x over previous
import functools

import jax
import jax.numpy as jnp
from jax import lax
from jax.experimental import pallas as pl
from jax.experimental.pallas import tpu as pltpu
from jax.experimental.pallas import tpu_sc as plsc

N = 10000
E = 320000
D = 128
NC = 2
NS = 16
NW = NC * NS
EPW = E // NW
B = 80
NCHUNK = EPW // B
SC_CH = 25
NSUP = NCHUNK // SC_CH
NPAD = 10240
NPW = NPAD // NS
FV = D // 16
BLK = 2000


def _sc_segment_sum(y, src3, dst3, ew2):
    mesh = plsc.VectorSubcoreMesh(core_axis_name="c", subcore_axis_name="s")

    @functools.partial(
        pl.kernel,
        out_type=jax.ShapeDtypeStruct((NC, NPAD, D), jnp.float32),
        mesh=mesh,
        scratch_types=[
            pltpu.VMEM((SC_CH, B), jnp.int32),
            pltpu.VMEM((SC_CH, B), jnp.int32),
            pltpu.VMEM((SC_CH, B), jnp.float32),
            pltpu.VMEM((B, D), jnp.float32),
            pltpu.VMEM((B, D), jnp.float32),
            pltpu.VMEM_SHARED((NPAD, D), jnp.float32),
            pltpu.SemaphoreType.DMA,
            pltpu.SemaphoreType.DMA,
        ],
    )
    def seg(y_hbm, src_hbm, dst_hbm, ew_hbm, out_hbm,
            src_v, dst_v, ew_v, rows0_v, rows1_v, acc_sh, sem0, sem1):
        rows = [rows0_v, rows1_v]
        gsem = [sem0, sem1]
        cid = lax.axis_index("c")
        sid = lax.axis_index("s")
        wid = sid * NC + cid

        z16 = jnp.zeros((16,), jnp.float32)

        def zrow(i, carry):
            for f in range(FV):
                rows0_v[i, pl.ds(f * 16, 16)] = z16
            return carry

        lax.fori_loop(0, B, zrow, 0)
        for z in range(NPW // B):
            pltpu.sync_copy(rows0_v,
                            acc_sh.at[pl.ds((sid * (NPW // B) + z) * B, B)])

        plsc.subcore_barrier()

        def sup(s, scarry):
            pltpu.sync_copy(src_hbm.at[wid, s], src_v)
            pltpu.sync_copy(dst_hbm.at[wid, s], dst_v)
            pltpu.sync_copy(ew_hbm.at[wid, s], ew_v)

            pltpu.async_copy(y_hbm.at[src_v.at[0]], rows[0], gsem[0])

            def chunk(c, carry):
                for k in range(2):
                    @pl.when(c % 2 == k)
                    def _(k=k):
                        cur_v = rows[k]
                        nk = 1 - k

                        pltpu.make_async_copy(
                            y_hbm.at[pl.ds(0, B)], cur_v, gsem[k]).wait()

                        @pl.when(c < SC_CH - 1)
                        def _():
                            pltpu.async_copy(
                                y_hbm.at[src_v.at[c + 1]], rows[nk], gsem[nk])

                        for g in range(B // 16):
                            ew16 = ew_v[c, pl.ds(g * 16, 16)]
                            for j in range(16):
                                w = jnp.broadcast_to(ew16[j], (16,))
                                i = g * 16 + j
                                for f in range(FV):
                                    cur_v[i, pl.ds(f * 16, 16)] = (
                                        cur_v[i, pl.ds(f * 16, 16)] * w)

                        pltpu.sync_copy(cur_v, acc_sh.at[dst_v.at[c]], add=True)

                return carry

            lax.fori_loop(0, SC_CH, chunk, 0)
            return scarry

        lax.fori_loop(0, NSUP, sup, 0)

        plsc.subcore_barrier()
        pltpu.sync_copy(acc_sh.at[pl.ds(sid * NPW, NPW)],
                        out_hbm.at[cid, pl.ds(sid * NPW, NPW)])

    return seg(y, src3, dst3, ew2)


def _dotT(a, w):
    return lax.dot_general(a, w, (((1,), (1,)), ((), ())),
                           preferred_element_type=jnp.float32)


def _layer1_body(p_ref, x_ref, wr_ref, wo_ref, b_ref, h_ref):
    aggr = p_ref[0] + p_ref[1]
    h_ref[...] = jnp.maximum(
        _dotT(aggr, wr_ref[...]) + b_ref[...] + _dotT(x_ref[...], wo_ref[...]), 0.0)


def _layer2_body(p_ref, h_ref, wr_ref, wo_ref, b_ref, wl_ref, bl_ref, o_ref):
    aggr = p_ref[0] + p_ref[1]
    g = jnp.maximum(
        _dotT(aggr, wr_ref[...]) + b_ref[...] + _dotT(h_ref[...], wo_ref[...]), 0.0)
    o_ref[...] = _dotT(g, wl_ref[...]) + bl_ref[...]


_row_spec = pl.BlockSpec((BLK, D), lambda i: (i, 0))
_w_spec = pl.BlockSpec((D, D), lambda i: (0, 0))
_b_spec = pl.BlockSpec((1, D), lambda i: (0, 0))
_p_spec = pl.BlockSpec((NC, BLK, D), lambda i: (0, i, 0))
_nd_f32 = jax.ShapeDtypeStruct((N, D), jnp.float32)


def _tc_layer1(parts, x, W_rel, b_rel, W_root):
    return pl.pallas_call(
        _layer1_body,
        grid=(N // BLK,),
        in_specs=[_p_spec, _row_spec, _w_spec, _w_spec, _b_spec],
        out_specs=_row_spec,
        out_shape=_nd_f32,
    )(parts, x, W_rel, W_root, b_rel.reshape(1, D))


def _tc_layer2(parts, h, W_rel, b_rel, W_root, W_lin, b_lin):
    return pl.pallas_call(
        _layer2_body,
        grid=(N // BLK,),
        in_specs=[_p_spec, _row_spec, _w_spec, _w_spec, _b_spec, _w_spec, _b_spec],
        out_specs=_row_spec,
        out_shape=_nd_f32,
    )(parts, h, W_rel, W_root, b_rel.reshape(1, D), W_lin, b_lin.reshape(1, D))


def kernel(x, edge_index, edge_weight,
           W_rel1, b_rel1, W_root1, W_rel2, b_rel2, W_root2, W_lin, b_lin):
    src3 = edge_index[0].astype(jnp.int32).reshape(NW, NSUP, SC_CH, B)
    dst3 = edge_index[1].astype(jnp.int32).reshape(NW, NSUP, SC_CH, B)
    ew2 = edge_weight.astype(jnp.float32).reshape(NW, NSUP, SC_CH, B)

    p1 = _sc_segment_sum(x, src3, dst3, ew2)
    h = _tc_layer1(p1, x, W_rel1, b_rel1, W_root1)
    p2 = _sc_segment_sum(h, src3, dst3, ew2)
    return _tc_layer2(p2, h, W_rel2, b_rel2, W_root2, W_lin, b_lin)


# speedup vs baseline: 2.5979x; 1.0328x over previous
import functools

import jax
import jax.numpy as jnp
from jax import lax
from jax.experimental import pallas as pl
from jax.experimental.pallas import tpu as pltpu
from jax.experimental.pallas import tpu_sc as plsc

N = 10000
E = 320000
D = 128
NC = 2
NS = 16
NW = NC * NS
EPW = E // NW
B = 80
NCHUNK = EPW // B
SC_CH = 25
NSUP = NCHUNK // SC_CH
NPAD = 10240
NPW = NPAD // NS
FV = D // 16
BLK = 2000


def _sc_segment_sum(y, src3, dst3, ew2):
    mesh = plsc.VectorSubcoreMesh(core_axis_name="c", subcore_axis_name="s")

    @functools.partial(
        pl.kernel,
        out_type=jax.ShapeDtypeStruct((NC, NPAD, D), jnp.float32),
        mesh=mesh,
        scratch_types=[
            pltpu.VMEM((SC_CH, B), jnp.int32),
            pltpu.VMEM((SC_CH, B), jnp.int32),
            pltpu.VMEM((SC_CH, B), jnp.float32),
            pltpu.VMEM((B, D), jnp.float32),
            pltpu.VMEM((B, D), jnp.float32),
            pltpu.VMEM_SHARED((NPAD, D), jnp.float32),
            pltpu.SemaphoreType.DMA,
            pltpu.SemaphoreType.DMA,
        ],
    )
    def seg(y_hbm, src_hbm, dst_hbm, ew_hbm, out_hbm,
            src_v, dst_v, ew_v, rows0_v, rows1_v, acc_sh, sem0, sem1):
        rows = [rows0_v, rows1_v]
        gsem = [sem0, sem1]
        cid = lax.axis_index("c")
        sid = lax.axis_index("s")
        wid = sid * NC + cid

        z16 = jnp.zeros((16,), jnp.float32)

        def zrow(i, carry):
            for f in range(FV):
                rows0_v[i, pl.ds(f * 16, 16)] = z16
            return carry

        lax.fori_loop(0, B, zrow, 0)
        zcps = [pltpu.async_copy(
                    rows0_v, acc_sh.at[pl.ds((sid * (NPW // B) + z) * B, B)],
                    sem1)
                for z in range(NPW // B)]
        for cp in zcps:
            cp.wait()

        plsc.subcore_barrier()

        def sup(s, scarry):
            e1 = pltpu.async_copy(src_hbm.at[wid, s], src_v, sem1)
            e2 = pltpu.async_copy(dst_hbm.at[wid, s], dst_v, sem1)
            e3 = pltpu.async_copy(ew_hbm.at[wid, s], ew_v, sem1)
            e1.wait()
            e2.wait()
            e3.wait()

            pltpu.async_copy(y_hbm.at[src_v.at[0]], rows[0], gsem[0])

            def chunk(c, carry):
                for k in range(2):
                    @pl.when(c % 2 == k)
                    def _(k=k):
                        cur_v = rows[k]
                        nk = 1 - k

                        pltpu.make_async_copy(
                            y_hbm.at[pl.ds(0, B)], cur_v, gsem[k]).wait()

                        @pl.when(c < SC_CH - 1)
                        def _():
                            pltpu.async_copy(
                                y_hbm.at[src_v.at[c + 1]], rows[nk], gsem[nk])

                        for g in range(B // 16):
                            ew16 = ew_v[c, pl.ds(g * 16, 16)]
                            for j in range(16):
                                w = jnp.broadcast_to(ew16[j], (16,))
                                i = g * 16 + j
                                for f in range(FV):
                                    cur_v[i, pl.ds(f * 16, 16)] = (
                                        cur_v[i, pl.ds(f * 16, 16)] * w)

                        pltpu.sync_copy(cur_v, acc_sh.at[dst_v.at[c]], add=True)

                return carry

            lax.fori_loop(0, SC_CH, chunk, 0)
            return scarry

        lax.fori_loop(0, NSUP, sup, 0)

        plsc.subcore_barrier()
        pltpu.sync_copy(acc_sh.at[pl.ds(sid * NPW, NPW)],
                        out_hbm.at[cid, pl.ds(sid * NPW, NPW)])

    return seg(y, src3, dst3, ew2)


def _dotT(a, w):
    return lax.dot_general(a, w, (((1,), (1,)), ((), ())),
                           preferred_element_type=jnp.float32)


def _layer1_body(p_ref, x_ref, wr_ref, wo_ref, b_ref, h_ref):
    aggr = p_ref[0] + p_ref[1]
    h_ref[...] = jnp.maximum(
        _dotT(aggr, wr_ref[...]) + b_ref[...] + _dotT(x_ref[...], wo_ref[...]), 0.0)


def _layer2_body(p_ref, h_ref, wr_ref, wo_ref, b_ref, wl_ref, bl_ref, o_ref):
    aggr = p_ref[0] + p_ref[1]
    g = jnp.maximum(
        _dotT(aggr, wr_ref[...]) + b_ref[...] + _dotT(h_ref[...], wo_ref[...]), 0.0)
    o_ref[...] = _dotT(g, wl_ref[...]) + bl_ref[...]


_row_spec = pl.BlockSpec((BLK, D), lambda i: (i, 0))
_w_spec = pl.BlockSpec((D, D), lambda i: (0, 0))
_b_spec = pl.BlockSpec((1, D), lambda i: (0, 0))
_p_spec = pl.BlockSpec((NC, BLK, D), lambda i: (0, i, 0))
_nd_f32 = jax.ShapeDtypeStruct((N, D), jnp.float32)


def _tc_layer1(parts, x, W_rel, b_rel, W_root):
    return pl.pallas_call(
        _layer1_body,
        grid=(N // BLK,),
        in_specs=[_p_spec, _row_spec, _w_spec, _w_spec, _b_spec],
        out_specs=_row_spec,
        out_shape=_nd_f32,
    )(parts, x, W_rel, W_root, b_rel.reshape(1, D))


def _tc_layer2(parts, h, W_rel, b_rel, W_root, W_lin, b_lin):
    return pl.pallas_call(
        _layer2_body,
        grid=(N // BLK,),
        in_specs=[_p_spec, _row_spec, _w_spec, _w_spec, _b_spec, _w_spec, _b_spec],
        out_specs=_row_spec,
        out_shape=_nd_f32,
    )(parts, h, W_rel, W_root, b_rel.reshape(1, D), W_lin, b_lin.reshape(1, D))


def kernel(x, edge_index, edge_weight,
           W_rel1, b_rel1, W_root1, W_rel2, b_rel2, W_root2, W_lin, b_lin):
    src3 = edge_index[0].astype(jnp.int32).reshape(NW, NSUP, SC_CH, B)
    dst3 = edge_index[1].astype(jnp.int32).reshape(NW, NSUP, SC_CH, B)
    ew2 = edge_weight.astype(jnp.float32).reshape(NW, NSUP, SC_CH, B)

    p1 = _sc_segment_sum(x, src3, dst3, ew2)
    h = _tc_layer1(p1, x, W_rel1, b_rel1, W_root1)
    p2 = _sc_segment_sum(h, src3, dst3, ew2)
    return _tc_layer2(p2, h, W_rel2, b_rel2, W_root2, W_lin, b_lin)
